# baseline (speedup 1.0000x reference)
import functools

import jax
import jax.numpy as jnp
from jax import lax
from jax.experimental import pallas as pl
from jax.experimental.pallas import tpu as pltpu

try:
    from jax.experimental.pallas import tpu_sc as plsc
    _HAS_SC = True
except ImportError:
    _HAS_SC = False


def _sc_gather(table, ids):
    n = ids.shape[0]
    d = table.shape[1]
    bpw = 8
    n_active = n // bpw
    mesh = plsc.VectorSubcoreMesh(core_axis_name="c", subcore_axis_name="s")
    info = plsc.get_sparse_core_info()
    nc = info.num_cores

    @functools.partial(
        pl.kernel,
        mesh=mesh,
        out_type=jax.ShapeDtypeStruct((n, d), jnp.float32),
        scratch_types=[
            pltpu.VMEM((bpw,), jnp.int32),
            pltpu.VMEM((bpw, d), jnp.float32),
            pltpu.SemaphoreType.DMA,
        ],
    )
    def gk(table_hbm, idx_hbm, out_hbm, idx_v, rows_v, sem):
        wid = lax.axis_index("s") * nc + lax.axis_index("c")

        @pl.when(wid < n_active)
        def _():
            base = wid * bpw
            pltpu.sync_copy(idx_hbm.at[pl.ds(base, bpw)], idx_v)
            pltpu.async_copy(table_hbm.at[idx_v], rows_v, sem).wait()
            pltpu.sync_copy(rows_v, out_hbm.at[pl.ds(base, bpw)])

    return gk(table, ids)


def _mlp_body(nl, fft, thr_ref, h0_ref, w1_ref, b1_ref, w2_ref, b2_ref,
              g_ref, lnb_ref, gw_ref, hout_ref, vloss_ref, used_ref,
              h_scr, ff_scr, acc_scr, vl_scr, us_scr):
    l = pl.program_id(0)
    sub = pl.program_id(1)

    @pl.when(jnp.logical_and(l == 0, sub == 0))
    def _init():
        h_scr[...] = h0_ref[...]
        vl_scr[0] = jnp.float32(0.0)
        us_scr[0] = jnp.int32(0)

    dq = fft

    @pl.when(sub < 4)
    def _up_proj():
        part = jnp.dot(h_scr[:, pl.ds(sub * dq, dq)], w1_ref[0],
                       preferred_element_type=jnp.float32)

        @pl.when(sub == 0)
        def _set():
            ff_scr[...] = part

        @pl.when(sub > 0)
        def _add():
            ff_scr[...] += part

        @pl.when(sub == 3)
        def _act():
            ff_scr[...] = jax.nn.gelu(ff_scr[...] + b1_ref[0, 0])

    @pl.when(jnp.logical_and(sub >= 4, sub < 7))
    def _down():
        q = sub - 4
        part = jnp.dot(ff_scr[:, pl.ds(q * 1024, 1024)], w2_ref[0],
                       preferred_element_type=jnp.float32)

        @pl.when(sub == 4)
        def _set():
            acc_scr[...] = part

        @pl.when(sub > 4)
        def _add():
            acc_scr[...] += part

    @pl.when(sub == 7)
    def _down_last():
        h = h_scr[...]
        x = h + acc_scr[...] + jnp.dot(
            ff_scr[:, pl.ds(3 * 1024, 1024)], w2_ref[0],
            preferred_element_type=jnp.float32) + b2_ref[0, 0]
        m = jnp.mean(x, axis=-1, keepdims=True)
        v = jnp.mean((x - m) ** 2, axis=-1, keepdims=True)
        h_new = (x - m) / jnp.sqrt(v + 1e-5) * g_ref[0, 0] + lnb_ref[0, 0]
        vl_scr[0] += jnp.mean((h_new - h) ** 2)
        conf = jax.nn.sigmoid(jnp.mean(jnp.sum(h_new * gw_ref[0, 0], axis=-1)))
        us_scr[0] += (conf < thr_ref[0]).astype(jnp.int32)
        h_scr[...] = h_new

    @pl.when(jnp.logical_and(l == nl - 1, sub == 7))
    def _finish():
        hout_ref[...] = h_scr[...]
        vloss_ref[0] = vl_scr[0] / nl
        used_ref[0] = us_scr[0]


def _mlp_stack(h0, thr, w1, b1, w2, b2, ln_g, ln_b, gate_w, *, interpret=False):
    nl, d, ff = w1.shape
    n = h0.shape[0]
    dq = d // 4
    f4 = ff // 4
    grid = (nl, 8)
    body = functools.partial(_mlp_body, nl, dq)
    return pl.pallas_call(
        body,
        grid=grid,
        in_specs=[
            pl.BlockSpec(memory_space=pltpu.SMEM),
            pl.BlockSpec((n, d), lambda l, s: (0, 0)),
            pl.BlockSpec((1, dq, ff),
                         lambda l, s: (l, jnp.minimum(s, 3), 0)),
            pl.BlockSpec((1, 1, ff), lambda l, s: (l, 0, 0)),
            pl.BlockSpec((1, f4, d),
                         lambda l, s: (l, jnp.maximum(s - 4, 0), 0)),
            pl.BlockSpec((1, 1, d), lambda l, s: (l, 0, 0)),
            pl.BlockSpec((1, 1, d), lambda l, s: (l, 0, 0)),
            pl.BlockSpec((1, 1, d), lambda l, s: (l, 0, 0)),
            pl.BlockSpec((1, 1, d), lambda l, s: (l, 0, 0)),
        ],
        out_specs=[
            pl.BlockSpec((n, d), lambda l, f: (0, 0)),
            pl.BlockSpec(memory_space=pltpu.SMEM),
            pl.BlockSpec(memory_space=pltpu.SMEM),
        ],
        out_shape=[
            jax.ShapeDtypeStruct((n, d), jnp.float32),
            jax.ShapeDtypeStruct((1,), jnp.float32),
            jax.ShapeDtypeStruct((1,), jnp.int32),
        ],
        scratch_shapes=[
            pltpu.VMEM((n, d), jnp.float32),
            pltpu.VMEM((n, ff), jnp.float32),
            pltpu.VMEM((n, d), jnp.float32),
            pltpu.SMEM((1,), jnp.float32),
            pltpu.SMEM((1,), jnp.int32),
        ],
        compiler_params=pltpu.CompilerParams(
            dimension_semantics=("arbitrary", "arbitrary")),
        interpret=interpret,
    )(thr, h0, w1, b1.reshape(nl, 1, ff), w2, b2.reshape(nl, 1, d),
      ln_g.reshape(nl, 1, d), ln_b.reshape(nl, 1, d), gate_w.reshape(nl, 1, d))


def _head_body(h_ref, w_ref, b_ref, out_ref):
    out_ref[...] = lax.dot_general(
        h_ref[...], w_ref[...],
        dimension_numbers=(((1,), (1,)), ((), ())),
        preferred_element_type=jnp.float32) + b_ref[0]


def _head(h, head_w, head_b, *, interpret=False):
    n, d = h.shape
    vocab = head_w.shape[1]
    wt = head_w.T
    vt = 4096
    nvt = pl.cdiv(vocab, vt)
    return pl.pallas_call(
        _head_body,
        grid=(nvt,),
        in_specs=[
            pl.BlockSpec((n, d), lambda v: (0, 0)),
            pl.BlockSpec((vt, d), lambda v: (v, 0)),
            pl.BlockSpec((1, vt), lambda v: (0, v)),
        ],
        out_specs=pl.BlockSpec((n, vt), lambda v: (0, v)),
        out_shape=jax.ShapeDtypeStruct((n, vocab), jnp.float32),
        compiler_params=pltpu.CompilerParams(
            dimension_semantics=("arbitrary",)),
        interpret=interpret,
    )(h, wt, head_b.reshape(1, vocab))


def kernel(input_ids, exit_threshold, embed_table, W1, b1, W2, b2,
           ln_g, ln_b, gate_w, head_w, head_b):
    b, s = input_ids.shape
    nl = W1.shape[0]
    ids = input_ids.reshape(-1).astype(jnp.int32)
    h0 = _sc_gather(embed_table, ids)
    thr = jnp.reshape(exit_threshold, (1,)).astype(jnp.float32)
    h, vloss, used = _mlp_stack(h0, thr, W1, b1, W2, b2, ln_g, ln_b, gate_w)
    logits = _head(h, head_w, head_b)
    layers_used = jnp.minimum(used[0] + 1, nl)
    return (logits.reshape(b, s, head_w.shape[1]), layers_used, vloss[0])


# speedup vs baseline: 1.0357x; 1.0357x over previous
import functools

import jax
import jax.numpy as jnp
from jax import lax
from jax.experimental import pallas as pl
from jax.experimental.pallas import tpu as pltpu

try:
    from jax.experimental.pallas import tpu_sc as plsc
    _HAS_SC = True
except ImportError:
    _HAS_SC = False


def _sc_gather(table, ids):
    n = ids.shape[0]
    d = table.shape[1]
    bpw = 8
    n_active = n // bpw
    mesh = plsc.VectorSubcoreMesh(core_axis_name="c", subcore_axis_name="s")
    info = plsc.get_sparse_core_info()
    nc = info.num_cores

    @functools.partial(
        pl.kernel,
        mesh=mesh,
        out_type=jax.ShapeDtypeStruct((n, d), jnp.float32),
        scratch_types=[
            pltpu.VMEM((bpw,), jnp.int32),
            pltpu.VMEM((bpw, d), jnp.float32),
            pltpu.SemaphoreType.DMA,
        ],
    )
    def gk(table_hbm, idx_hbm, out_hbm, idx_v, rows_v, sem):
        wid = lax.axis_index("s") * nc + lax.axis_index("c")

        @pl.when(wid < n_active)
        def _():
            base = wid * bpw
            pltpu.sync_copy(idx_hbm.at[pl.ds(base, bpw)], idx_v)
            pltpu.async_copy(table_hbm.at[idx_v], rows_v, sem).wait()
            pltpu.sync_copy(rows_v, out_hbm.at[pl.ds(base, bpw)])

    return gk(table, ids)


def _mlp_body(nl, fft, thr_ref, h0_ref, w1_ref, b1_ref, w2_ref, b2_ref,
              g_ref, lnb_ref, gw_ref, hout_ref, vloss_ref, used_ref,
              h_scr, ff_scr, acc_scr, vl_scr, us_scr):
    l = pl.program_id(0)
    sub = pl.program_id(1)

    @pl.when(jnp.logical_and(l == 0, sub == 0))
    def _init():
        h_scr[...] = h0_ref[...]
        vl_scr[0] = jnp.float32(0.0)
        us_scr[0] = jnp.int32(0)

    @pl.when(sub == 0)
    def _up_proj():
        h = h_scr[...]
        ff_scr[...] = jax.nn.gelu(
            jnp.dot(h, w1_ref[0], preferred_element_type=jnp.float32)
            + b1_ref[0, 0])
        acc_scr[...] = jnp.dot(ff_scr[:, pl.ds(0, fft)], w2_ref[0],
                               preferred_element_type=jnp.float32)

    @pl.when(sub == 1)
    def _down_b():
        h = h_scr[...]
        ffb = ff_scr[:, pl.ds(fft, fft)]
        x = h + acc_scr[...] + jnp.dot(
            ffb, w2_ref[0], preferred_element_type=jnp.float32) + b2_ref[0, 0]
        m = jnp.mean(x, axis=-1, keepdims=True)
        v = jnp.mean((x - m) ** 2, axis=-1, keepdims=True)
        h_new = (x - m) / jnp.sqrt(v + 1e-5) * g_ref[0, 0] + lnb_ref[0, 0]
        vl_scr[0] += jnp.mean((h_new - h) ** 2)
        conf = jax.nn.sigmoid(jnp.mean(jnp.sum(h_new * gw_ref[0, 0], axis=-1)))
        us_scr[0] += (conf < thr_ref[0]).astype(jnp.int32)
        h_scr[...] = h_new

    @pl.when(jnp.logical_and(l == nl - 1, sub == 1))
    def _finish():
        hout_ref[...] = h_scr[...]
        vloss_ref[0] = vl_scr[0] / nl
        used_ref[0] = us_scr[0]


def _mlp_stack(h0, thr, w1, b1, w2, b2, ln_g, ln_b, gate_w, *, interpret=False):
    nl, d, ff = w1.shape
    n = h0.shape[0]
    fft = ff // 2
    grid = (nl, 2)
    body = functools.partial(_mlp_body, nl, fft)
    return pl.pallas_call(
        body,
        grid=grid,
        in_specs=[
            pl.BlockSpec(memory_space=pltpu.SMEM),
            pl.BlockSpec((n, d), lambda l, s: (0, 0)),
            pl.BlockSpec((1, d, ff), lambda l, s: (l, 0, 0)),
            pl.BlockSpec((1, 1, ff), lambda l, s: (l, 0, 0)),
            pl.BlockSpec((1, fft, d), lambda l, s: (l, s, 0)),
            pl.BlockSpec((1, 1, d), lambda l, s: (l, 0, 0)),
            pl.BlockSpec((1, 1, d), lambda l, s: (l, 0, 0)),
            pl.BlockSpec((1, 1, d), lambda l, s: (l, 0, 0)),
            pl.BlockSpec((1, 1, d), lambda l, s: (l, 0, 0)),
        ],
        out_specs=[
            pl.BlockSpec((n, d), lambda l, f: (0, 0)),
            pl.BlockSpec(memory_space=pltpu.SMEM),
            pl.BlockSpec(memory_space=pltpu.SMEM),
        ],
        out_shape=[
            jax.ShapeDtypeStruct((n, d), jnp.float32),
            jax.ShapeDtypeStruct((1,), jnp.float32),
            jax.ShapeDtypeStruct((1,), jnp.int32),
        ],
        scratch_shapes=[
            pltpu.VMEM((n, d), jnp.float32),
            pltpu.VMEM((n, ff), jnp.float32),
            pltpu.VMEM((n, d), jnp.float32),
            pltpu.SMEM((1,), jnp.float32),
            pltpu.SMEM((1,), jnp.int32),
        ],
        compiler_params=pltpu.CompilerParams(
            dimension_semantics=("arbitrary", "arbitrary")),
        interpret=interpret,
    )(thr, h0, w1, b1.reshape(nl, 1, ff), w2, b2.reshape(nl, 1, d),
      ln_g.reshape(nl, 1, d), ln_b.reshape(nl, 1, d), gate_w.reshape(nl, 1, d))


def _head_body(h_ref, w_ref, b_ref, out_ref):
    out_ref[...] = lax.dot_general(
        h_ref[...], w_ref[...],
        dimension_numbers=(((1,), (1,)), ((), ())),
        preferred_element_type=jnp.float32) + b_ref[0]


def _head(h, head_w, head_b, *, interpret=False):
    n, d = h.shape
    vocab = head_w.shape[1]
    wt = head_w.T
    vt = 4352
    nvt = pl.cdiv(vocab, vt)
    return pl.pallas_call(
        _head_body,
        grid=(nvt,),
        in_specs=[
            pl.BlockSpec((n, d), lambda v: (0, 0)),
            pl.BlockSpec((vt, d), lambda v: (v, 0)),
            pl.BlockSpec((1, vt), lambda v: (0, v)),
        ],
        out_specs=pl.BlockSpec((n, vt), lambda v: (0, v)),
        out_shape=jax.ShapeDtypeStruct((n, vocab), jnp.float32),
        compiler_params=pltpu.CompilerParams(
            dimension_semantics=("arbitrary",)),
        interpret=interpret,
    )(h, wt, head_b.reshape(1, vocab))


def kernel(input_ids, exit_threshold, embed_table, W1, b1, W2, b2,
           ln_g, ln_b, gate_w, head_w, head_b):
    b, s = input_ids.shape
    nl = W1.shape[0]
    ids = input_ids.reshape(-1).astype(jnp.int32)
    h0 = _sc_gather(embed_table, ids)
    thr = jnp.reshape(exit_threshold, (1,)).astype(jnp.float32)
    h, vloss, used = _mlp_stack(h0, thr, W1, b1, W2, b2, ln_g, ln_b, gate_w)
    logits = _head(h, head_w, head_b)
    layers_used = jnp.minimum(used[0] + 1, nl)
    return (logits.reshape(b, s, head_w.shape[1]), layers_used, vloss[0])
